# Initial kernel scaffold; baseline (speedup 1.0000x reference)
import functools

import jax
import jax.numpy as jnp
from jax import lax
from jax.experimental import pallas as pl
from jax.experimental.pallas import tpu as pltpu
from jax.experimental.pallas import tpu_sc as plsc

_EMBED = 64
_BATCH = 16384
_HIST = 50
_B = _BATCH * _HIST

_NC = 2
_NS = 16
_NW = _NC * _NS
_C = 128
_ROWS_PER_W = _B // _NW
_N_CHUNKS = _ROWS_PER_W // _C


def _build_kernel():
    mesh = plsc.VectorSubcoreMesh(core_axis_name="c", subcore_axis_name="s")

    @functools.partial(
        pl.kernel,
        mesh=mesh,
        out_type=jax.ShapeDtypeStruct((_B, _EMBED), jnp.float32),
        scratch_types=[
            pltpu.VMEM((_N_CHUNKS, _C), jnp.int32),
            pltpu.VMEM((_C, _EMBED), jnp.float32),
            pltpu.SemaphoreType.DMA,
        ],
    )
    def gather_kernel(tok_hbm, tab_hbm, out_hbm, idx_v, rows_v, sem):
        wid = lax.axis_index("s") * _NC + lax.axis_index("c")
        base = wid * _ROWS_PER_W
        pltpu.sync_copy(tok_hbm.at[wid], idx_v)

        def step(j, carry):
            pltpu.async_copy(tab_hbm.at[idx_v.at[j]], rows_v, sem).wait()
            pltpu.sync_copy(rows_v, out_hbm.at[pl.ds(base + j * _C, _C)])
            return carry

        lax.fori_loop(0, _N_CHUNKS, step, 0)

    return gather_kernel


_KERNEL = _build_kernel()


def kernel(token, embed):
    tok = token.reshape(_NW, _N_CHUNKS, _C).astype(jnp.int32)
    out = _KERNEL(tok, embed)
    return lax.stop_gradient(out.reshape(_BATCH, _HIST, _EMBED))


# speedup vs baseline: 1.6845x; 1.6845x over previous
import functools

import jax
import jax.numpy as jnp
from jax import lax
from jax.experimental import pallas as pl
from jax.experimental.pallas import tpu as pltpu
from jax.experimental.pallas import tpu_sc as plsc

_EMBED = 64
_BATCH = 16384
_HIST = 50
_B = _BATCH * _HIST

_NC = 2
_NS = 16
_NW = _NC * _NS
_C = 128
_ROWS_PER_W = _B // _NW
_N_CHUNKS = _ROWS_PER_W // _C


def _build_kernel():
    mesh = plsc.VectorSubcoreMesh(core_axis_name="c", subcore_axis_name="s")

    @functools.partial(
        pl.kernel,
        mesh=mesh,
        out_type=jax.ShapeDtypeStruct((_B, _EMBED), jnp.float32),
        scratch_types=[
            pltpu.VMEM((_N_CHUNKS, _C), jnp.int32),
            pltpu.VMEM((_C, _EMBED), jnp.float32),
            pltpu.SemaphoreType.DMA,
        ],
        compiler_params=pltpu.CompilerParams(use_tc_tiling_on_sc=False),
    )
    def gather_kernel(tok_hbm, tab_hbm, out_hbm, idx_v, rows_v, sem):
        wid = lax.axis_index("s") * _NC + lax.axis_index("c")
        base = wid * _ROWS_PER_W
        pltpu.sync_copy(tok_hbm.at[wid], idx_v)

        def step(j, carry):
            pltpu.async_copy(tab_hbm.at[idx_v.at[j]], rows_v, sem).wait()
            pltpu.sync_copy(rows_v, out_hbm.at[pl.ds(base + j * _C, _C)])
            return carry

        lax.fori_loop(0, _N_CHUNKS, step, 0)

    return gather_kernel


_KERNEL = _build_kernel()


def kernel(token, embed):
    tok = token.reshape(_NW, _N_CHUNKS, _C).astype(jnp.int32)
    out = _KERNEL(tok, embed)
    return lax.stop_gradient(out.reshape(_BATCH, _HIST, _EMBED))


# speedup vs baseline: 1.8767x; 1.1141x over previous
import functools

import jax
import jax.numpy as jnp
from jax import lax
from jax.experimental import pallas as pl
from jax.experimental.pallas import tpu as pltpu
from jax.experimental.pallas import tpu_sc as plsc

_EMBED = 64
_BATCH = 16384
_HIST = 50
_B = _BATCH * _HIST

_NC = 2
_NS = 16
_NW = _NC * _NS
_C = 128
_ROWS_PER_W = _B // _NW
_N_CHUNKS = _ROWS_PER_W // _C


def _build_kernel():
    mesh = plsc.VectorSubcoreMesh(core_axis_name="c", subcore_axis_name="s")

    NBUF = 8

    @functools.partial(
        pl.kernel,
        mesh=mesh,
        out_type=jax.ShapeDtypeStruct((_B, _EMBED), jnp.float32),
        scratch_types=[
            pltpu.VMEM((_N_CHUNKS, _C), jnp.int32),
            pltpu.VMEM((NBUF, _C, _EMBED), jnp.float32),
            pltpu.SemaphoreType.DMA((NBUF,)),
            pltpu.SemaphoreType.DMA((NBUF,)),
        ],
        compiler_params=pltpu.CompilerParams(use_tc_tiling_on_sc=False),
    )
    def gather_kernel(tok_hbm, tab_hbm, out_hbm, idx_v, rows_v, gsem, osem):
        wid = lax.axis_index("s") * _NC + lax.axis_index("c")
        base = wid * _ROWS_PER_W
        pltpu.sync_copy(tok_hbm.at[wid], idx_v)

        for b in range(NBUF - 1):
            pltpu.async_copy(tab_hbm.at[idx_v.at[b]], rows_v.at[b], gsem.at[b])

        def step(j, carry):
            b = lax.rem(j, NBUF)
            jn = j + NBUF - 1
            bn = lax.rem(jn, NBUF)

            @pl.when(jn < _N_CHUNKS)
            def _start_gather():
                @pl.when(jn >= NBUF)
                def _wait_out():
                    pltpu.make_async_copy(
                        rows_v.at[bn], out_hbm.at[pl.ds(0, _C)], osem.at[bn]
                    ).wait()

                pltpu.async_copy(tab_hbm.at[idx_v.at[jn]], rows_v.at[bn], gsem.at[bn])

            pltpu.make_async_copy(
                tab_hbm.at[pl.ds(0, _C)], rows_v.at[b], gsem.at[b]
            ).wait()
            pltpu.async_copy(rows_v.at[b], out_hbm.at[pl.ds(base + j * _C, _C)], osem.at[b])
            return carry

        lax.fori_loop(0, _N_CHUNKS, step, 0)

        for i in range(NBUF):
            b = (_N_CHUNKS - NBUF + i) % NBUF
            pltpu.make_async_copy(
                rows_v.at[b], out_hbm.at[pl.ds(0, _C)], osem.at[b]
            ).wait()

    return gather_kernel


_KERNEL = _build_kernel()


def kernel(token, embed):
    tok = token.reshape(_NW, _N_CHUNKS, _C).astype(jnp.int32)
    out = _KERNEL(tok, embed)
    return lax.stop_gradient(out.reshape(_BATCH, _HIST, _EMBED))
